# Initial kernel scaffold; baseline (speedup 1.0000x reference)
import functools

import jax
import jax.numpy as jnp
from jax import lax
from jax.experimental import pallas as pl
from jax.experimental.pallas import tpu as pltpu
from jax.experimental.pallas import tpu_sc as plsc

MARGIN = 0.2
BATCH = 16384
DIM = 128
N_CLASSES = 1000

NC = 2
NS = 16
NW = NC * NS
TRIP_PER_W = BATCH // NW
ROWS_PER_W = 3 * TRIP_PER_W
CHUNK = 512
NCHUNK = ROWS_PER_W // CHUNK
BETA_PAD = 1024

R = 2048
NB = BATCH // R


def _sc_gather(batch, idx_all, labels, beta_p):
    mesh = plsc.VectorSubcoreMesh(core_axis_name="c", subcore_axis_name="s")

    @functools.partial(
        pl.kernel,
        out_type=(
            jax.ShapeDtypeStruct((3 * BATCH, DIM), jnp.float32),
            jax.ShapeDtypeStruct((BATCH,), jnp.float32),
        ),
        mesh=mesh,
        scratch_types=[
            pltpu.VMEM((CHUNK,), jnp.int32),
            pltpu.VMEM((CHUNK, DIM), jnp.float32),
            pltpu.VMEM((TRIP_PER_W,), jnp.int32),
            pltpu.VMEM((BATCH,), jnp.int32),
            pltpu.VMEM((BETA_PAD,), jnp.float32),
            pltpu.VMEM((TRIP_PER_W,), jnp.float32),
        ],
    )
    def k(batch_hbm, idx_hbm, labels_hbm, beta_hbm, rows_out, beta_t_out,
          idxc_v, rows_v, t0_v, labels_v, beta_v, bt_v):
        wid = lax.axis_index("s") * NC + lax.axis_index("c")

        for c in range(NCHUNK):
            base = wid * ROWS_PER_W + c * CHUNK
            pltpu.sync_copy(idx_hbm.at[pl.ds(base, CHUNK)], idxc_v)
            pltpu.sync_copy(batch_hbm.at[idxc_v], rows_v)
            pltpu.sync_copy(rows_v, rows_out.at[pl.ds(base, CHUNK)])

        tbase = wid * TRIP_PER_W
        pltpu.sync_copy(idx_hbm.at[pl.ds(tbase, TRIP_PER_W)], t0_v)
        pltpu.sync_copy(labels_hbm, labels_v)
        pltpu.sync_copy(beta_hbm, beta_v)

        @pl.loop(0, TRIP_PER_W // 16)
        def _(i):
            t0 = t0_v[pl.ds(i * 16, 16)]
            la = plsc.load_gather(labels_v, [t0])
            bt = plsc.load_gather(beta_v, [la])
            bt_v[pl.ds(i * 16, 16)] = bt

        pltpu.sync_copy(bt_v, beta_t_out.at[pl.ds(tbase, TRIP_PER_W)])

    return k(batch, idx_all, labels, beta_p)


def _tc_reduce_body(a_ref, p_ref, n_ref, bt_ref, out_ref, acc_ref):
    i = pl.program_id(0)

    @pl.when(i == 0)
    def _():
        acc_ref[0] = 0.0
        acc_ref[1] = 0.0

    a = a_ref[...]
    p = p_ref[...]
    n = n_ref[...]
    bt = bt_ref[0, 0]
    d_ap = jnp.sqrt(jnp.sum((a - p) ** 2, axis=1) + 1e-8)
    d_an = jnp.sqrt(jnp.sum((a - n) ** 2, axis=1) + 1e-8)
    pos = jnp.maximum(d_ap - bt + MARGIN, 0.0)
    neg = jnp.maximum(bt - d_an + MARGIN, 0.0)
    acc_ref[0] += jnp.sum(pos + neg)
    acc_ref[1] += jnp.sum((pos > 0.0).astype(jnp.float32)
                          + (neg > 0.0).astype(jnp.float32))

    @pl.when(i == NB - 1)
    def _():
        tot = acc_ref[0]
        cnt = acc_ref[1]
        out_ref[0, 0] = jnp.where(cnt == 0.0, tot, tot / jnp.maximum(cnt, 1.0))


def _tc_reduce(rows, beta_t):
    bt3 = beta_t.reshape(NB, 1, R)
    return pl.pallas_call(
        _tc_reduce_body,
        grid=(NB,),
        in_specs=[
            pl.BlockSpec((R, DIM), lambda i: (i, 0)),
            pl.BlockSpec((R, DIM), lambda i: (i + NB, 0)),
            pl.BlockSpec((R, DIM), lambda i: (i + 2 * NB, 0)),
            pl.BlockSpec((1, 1, R), lambda i: (i, 0, 0)),
        ],
        out_specs=pl.BlockSpec((1, 1), lambda i: (0, 0)),
        out_shape=jax.ShapeDtypeStruct((1, 1), jnp.float32),
        scratch_shapes=[pltpu.SMEM((2,), jnp.float32)],
    )(rows, rows, rows, bt3)


def kernel(batch, beta, labels, triplets):
    idx_all = jnp.transpose(triplets).reshape(3 * BATCH)
    beta_p = jnp.pad(beta, (0, BETA_PAD - beta.shape[0]))
    rows, beta_t = _sc_gather(batch, idx_all, labels, beta_p)
    loss = _tc_reduce(rows, beta_t)
    return loss[0, 0]


# speedup vs baseline: 1.9922x; 1.9922x over previous
import dataclasses
import functools

import jax
import jax.numpy as jnp
from jax import lax
from jax.experimental import pallas as pl
from jax.experimental.pallas import tpu as pltpu
from jax.experimental.pallas import tpu_sc as plsc

MARGIN = 0.2
BATCH = 16384
DIM = 128
N_CLASSES = 1000

NC = 2
NS = 16
NW = NC * NS
TRIP_PER_W = BATCH // NW
ROWS_PER_W = 3 * TRIP_PER_W
CHUNK = 512
NCHUNK = ROWS_PER_W // CHUNK
BETA_PAD = 1024

R = 2048
NB = BATCH // R


def _sc_gather(batch, idx_all, labels, beta_p):
    mesh = plsc.VectorSubcoreMesh(core_axis_name="c", subcore_axis_name="s")
    cp = pltpu.CompilerParams()
    if "needs_layout_passes" in pltpu.CompilerParams.__dataclass_fields__:
        cp = dataclasses.replace(cp, needs_layout_passes=False)

    @functools.partial(
        pl.kernel,
        compiler_params=cp,
        out_type=(
            jax.ShapeDtypeStruct((3 * BATCH, DIM), jnp.float32),
            jax.ShapeDtypeStruct((BATCH,), jnp.float32),
        ),
        mesh=mesh,
        scratch_types=[
            pltpu.VMEM((CHUNK,), jnp.int32),
            pltpu.VMEM((CHUNK, DIM), jnp.float32),
            pltpu.VMEM((TRIP_PER_W,), jnp.int32),
            pltpu.VMEM((BATCH,), jnp.int32),
            pltpu.VMEM((BETA_PAD,), jnp.float32),
            pltpu.VMEM((TRIP_PER_W,), jnp.float32),
        ],
    )
    def k(batch_hbm, idx_hbm, labels_hbm, beta_hbm, rows_out, beta_t_out,
          idxc_v, rows_v, t0_v, labels_v, beta_v, bt_v):
        wid = lax.axis_index("s") * NC + lax.axis_index("c")

        for c in range(NCHUNK):
            base = wid * ROWS_PER_W + c * CHUNK
            pltpu.sync_copy(idx_hbm.at[pl.ds(base, CHUNK)], idxc_v)
            pltpu.sync_copy(batch_hbm.at[idxc_v], rows_v)
            pltpu.sync_copy(rows_v, rows_out.at[pl.ds(base, CHUNK)])

        tbase = wid * TRIP_PER_W
        pltpu.sync_copy(idx_hbm.at[pl.ds(tbase, TRIP_PER_W)], t0_v)
        pltpu.sync_copy(labels_hbm, labels_v)
        pltpu.sync_copy(beta_hbm, beta_v)

        @pl.loop(0, TRIP_PER_W // 16)
        def _(i):
            t0 = t0_v[pl.ds(i * 16, 16)]
            la = plsc.load_gather(labels_v, [t0])
            bt = plsc.load_gather(beta_v, [la])
            bt_v[pl.ds(i * 16, 16)] = bt

        pltpu.sync_copy(bt_v, beta_t_out.at[pl.ds(tbase, TRIP_PER_W)])

    return k(batch, idx_all, labels, beta_p)


def _tc_reduce_body(a_ref, p_ref, n_ref, bt_ref, out_ref, acc_ref):
    i = pl.program_id(0)

    @pl.when(i == 0)
    def _():
        acc_ref[0] = 0.0
        acc_ref[1] = 0.0

    a = a_ref[...]
    p = p_ref[...]
    n = n_ref[...]
    bt = bt_ref[0, 0]
    d_ap = jnp.sqrt(jnp.sum((a - p) ** 2, axis=1) + 1e-8)
    d_an = jnp.sqrt(jnp.sum((a - n) ** 2, axis=1) + 1e-8)
    pos = jnp.maximum(d_ap - bt + MARGIN, 0.0)
    neg = jnp.maximum(bt - d_an + MARGIN, 0.0)
    acc_ref[0] += jnp.sum(pos + neg)
    acc_ref[1] += jnp.sum((pos > 0.0).astype(jnp.float32)
                          + (neg > 0.0).astype(jnp.float32))

    @pl.when(i == NB - 1)
    def _():
        tot = acc_ref[0]
        cnt = acc_ref[1]
        out_ref[0, 0] = jnp.where(cnt == 0.0, tot, tot / jnp.maximum(cnt, 1.0))


def _tc_reduce(rows, beta_t):
    bt3 = beta_t.reshape(NB, 1, R)
    return pl.pallas_call(
        _tc_reduce_body,
        grid=(NB,),
        in_specs=[
            pl.BlockSpec((R, DIM), lambda i: (i, 0)),
            pl.BlockSpec((R, DIM), lambda i: (i + NB, 0)),
            pl.BlockSpec((R, DIM), lambda i: (i + 2 * NB, 0)),
            pl.BlockSpec((1, 1, R), lambda i: (i, 0, 0)),
        ],
        out_specs=pl.BlockSpec(memory_space=pltpu.SMEM),
        out_shape=jax.ShapeDtypeStruct((1, 1), jnp.float32),
        scratch_shapes=[pltpu.SMEM((2,), jnp.float32)],
    )(rows, rows, rows, bt3)


def kernel(batch, beta, labels, triplets):
    idx_all = jnp.transpose(triplets).reshape(3 * BATCH)
    beta_p = jnp.pad(beta, (0, BETA_PAD - beta.shape[0]))
    rows, beta_t = _sc_gather(batch, idx_all, labels, beta_p)
    loss = _tc_reduce(rows, beta_t)
    return loss[0, 0]
